# baseline (speedup 1.0000x reference)
import jax
import jax.numpy as jnp
from jax import lax
from jax.experimental import pallas as pl
from jax.experimental.pallas import tpu as pltpu

_INC = 1024
_INN = 512
_OUTC = 2048
_OUTN = 64
_D = 4
_N = 32
_NB = 4


def _fgl_body(x_ref, a_ref, mask_ref, w2_ref, ctv2_ref, ctg2_ref, ctb2_ref,
              b128_ref, y_ref, s_ref, wt_ref, wn2_ref, sc_ref):
    step = pl.program_id(0)

    @pl.when(step == 0)
    def _init():
        iota = lax.broadcasted_iota(jnp.int32, (_OUTN, _INN), 1)
        st = jnp.zeros((_OUTN, _INN), jnp.float32)
        for d in range(_D):
            st = st + jnp.where(iota == a_ref[:, d:d + 1], mask_ref[:, d:d + 1], 0.0)
        s_ref[...] = st.T
        wt_ref[...] = w2_ref[...].T
        v = ctv2_ref[...]
        ve, vo = v[:, :_INC], v[:, _INC:]
        se = ctg2_ref[:, 0:1] * jax.lax.rsqrt(jnp.sum(ve * ve, axis=1, keepdims=True))
        so = ctg2_ref[:, 1:2] * jax.lax.rsqrt(jnp.sum(vo * vo, axis=1, keepdims=True))
        wn2_ref[:, :_INC] = ve * se
        wn2_ref[:, _INC:] = vo * so

    t_all = jnp.dot(x_ref[...].reshape(_NB * _INC, _INN), s_ref[...],
                    preferred_element_type=jnp.float32)
    wt = wt_ref[...]
    for q in range(_NB):
        sc_ref[:, _OUTN * q:_OUTN * (q + 1)] = t_all[_INC * q:_INC * (q + 1), :] * wt

    scaled = sc_ref[...]
    m2a = jnp.dot(wn2_ref[:, :_INC], scaled, preferred_element_type=jnp.float32)
    m2b = jnp.dot(wn2_ref[:, _INC:], scaled, preferred_element_type=jnp.float32)
    badd_e = b128_ref[:, :_OUTN] + ctb2_ref[:, 0:1]
    badd_o = b128_ref[:, _OUTN:] + ctb2_ref[:, 1:2]
    for q in range(_NB):
        y_ref[q, :, :_OUTN] = m2a[:, _OUTN * q:_OUTN * (q + 1)] + badd_e
        y_ref[q, :, _OUTN:] = m2b[:, _OUTN * q:_OUTN * (q + 1)] + badd_o


def kernel(x, A, mask, weight, ct_v, ct_g, ct_b, bias):
    a2 = A.astype(jnp.int32)
    mask2 = mask.reshape(_OUTN, _D)
    w2 = weight.reshape(_OUTN, _INC)
    ctv2 = ct_v.reshape(_OUTC // 2, 2 * _INC)
    ctg2 = ct_g.reshape(_OUTC // 2, 2)
    ctb2 = ct_b.reshape(_OUTC // 2, 2)
    b128 = bias.reshape(_OUTC // 2, 2 * _OUTN)

    y128 = pl.pallas_call(
        _fgl_body,
        grid=(_N // _NB,),
        in_specs=[
            pl.BlockSpec((_NB, _INC, _INN), lambda n: (n, 0, 0)),
            pl.BlockSpec((_OUTN, _D), lambda n: (0, 0)),
            pl.BlockSpec((_OUTN, _D), lambda n: (0, 0)),
            pl.BlockSpec((_OUTN, _INC), lambda n: (0, 0)),
            pl.BlockSpec((_OUTC // 2, 2 * _INC), lambda n: (0, 0)),
            pl.BlockSpec((_OUTC // 2, 2), lambda n: (0, 0)),
            pl.BlockSpec((_OUTC // 2, 2), lambda n: (0, 0)),
            pl.BlockSpec((_OUTC // 2, 2 * _OUTN), lambda n: (0, 0)),
        ],
        out_specs=pl.BlockSpec((_NB, _OUTC // 2, 2 * _OUTN), lambda n: (n, 0, 0)),
        out_shape=jax.ShapeDtypeStruct((_N, _OUTC // 2, 2 * _OUTN), jnp.float32),
        scratch_shapes=[
            pltpu.VMEM((_INN, _OUTN), jnp.float32),
            pltpu.VMEM((_INC, _OUTN), jnp.float32),
            pltpu.VMEM((_OUTC // 2, 2 * _INC), jnp.float32),
            pltpu.VMEM((_INC, _NB * _OUTN), jnp.float32),
        ],
    )(x, a2, mask2, w2, ctv2, ctg2, ctb2, b128)
    return y128.reshape(_N, _OUTC, _OUTN)


# speedup vs baseline: 1.6932x; 1.6932x over previous
import jax
import jax.numpy as jnp
from jax import lax
from jax.experimental import pallas as pl
from jax.experimental.pallas import tpu as pltpu

_INC = 1024
_INN = 512
_OUTC = 2048
_OUTN = 64
_D = 4
_N = 32
_NB = 4


def _fgl_body(x_ref, a_ref, mask_ref, w2_ref, ctv_ref, pack_ref,
              y_ref, s_ref, wt_ref, wn_ref, sc_ref, badd_ref):
    step = pl.program_id(0)

    @pl.when(step == 0)
    def _init():
        iota = lax.broadcasted_iota(jnp.int32, (_OUTN, _INN), 1)
        st = jnp.zeros((_OUTN, _INN), jnp.float32)
        for d in range(_D):
            st = st + jnp.where(iota == a_ref[:, d:d + 1], mask_ref[:, d:d + 1], 0.0)
        s_ref[...] = st.T
        wt_ref[...] = w2_ref[...].T
        v = ctv_ref[...]
        ctg_col = pack_ref[:, _OUTN:_OUTN + 1]
        ctb_col = pack_ref[:, _OUTN + 1:_OUTN + 2]
        scale = ctg_col * jax.lax.rsqrt(jnp.sum(v * v, axis=1, keepdims=True))
        wn_ref[...] = v * scale
        b = pack_ref[:, :_OUTN] + ctb_col
        badd_ref[...] = jnp.concatenate([b, b, b, b], axis=1)

    t_all = jnp.dot(x_ref[...].reshape(_NB * _INC, _INN), s_ref[...],
                    preferred_element_type=jnp.float32)
    wt = wt_ref[...]
    for q in range(_NB):
        sc_ref[:, _OUTN * q:_OUTN * (q + 1)] = t_all[_INC * q:_INC * (q + 1), :] * wt

    m2 = jnp.dot(wn_ref[...], sc_ref[...], preferred_element_type=jnp.float32)
    y_ref[...] = m2 + badd_ref[...]


def kernel(x, A, mask, weight, ct_v, ct_g, ct_b, bias):
    a2 = A.astype(jnp.int32)
    mask2 = mask.reshape(_OUTN, _D)
    w2 = weight.reshape(_OUTN, _INC)
    pack = jnp.concatenate(
        [bias, ct_g[:, None], ct_b[:, None],
         jnp.zeros((_OUTC, 128 - _OUTN - 2), jnp.float32)], axis=1)

    ay = pl.pallas_call(
        _fgl_body,
        grid=(_N // _NB,),
        in_specs=[
            pl.BlockSpec((_NB, _INC, _INN), lambda n: (n, 0, 0)),
            pl.BlockSpec((_OUTN, _D), lambda n: (0, 0)),
            pl.BlockSpec((_OUTN, _D), lambda n: (0, 0)),
            pl.BlockSpec((_OUTN, _INC), lambda n: (0, 0)),
            pl.BlockSpec((_OUTC, _INC), lambda n: (0, 0)),
            pl.BlockSpec((_OUTC, 128), lambda n: (0, 0)),
        ],
        out_specs=pl.BlockSpec((_OUTC, _NB * _OUTN), lambda n: (0, n)),
        out_shape=jax.ShapeDtypeStruct((_OUTC, _N * _OUTN), jnp.float32),
        scratch_shapes=[
            pltpu.VMEM((_INN, _OUTN), jnp.float32),
            pltpu.VMEM((_INC, _OUTN), jnp.float32),
            pltpu.VMEM((_OUTC, _INC), jnp.float32),
            pltpu.VMEM((_INC, _NB * _OUTN), jnp.float32),
            pltpu.VMEM((_OUTC, _NB * _OUTN), jnp.float32),
        ],
    )(x, a2, mask2, w2, ct_v, pack)
    return ay.reshape(_OUTC, _N, _OUTN).transpose(1, 0, 2)
